# Initial kernel scaffold; baseline (speedup 1.0000x reference)
import jax
import jax.numpy as jnp
from jax.experimental import pallas as pl


def kernel(x, w, b, gamma, beta):
    raise NotImplementedError("write your pallas kernel here")


# speedup vs baseline: 1.1207x; 1.1207x over previous
import functools
import math

import jax
import jax.numpy as jnp
from jax.experimental import pallas as pl
from jax.experimental.pallas import tpu as pltpu

_EPS = 1e-5


def _pick_tile(bs, nch):
    period = nch * 8 // math.gcd(nch, 8)
    for t in (2000, 1600, 1000, 800, 600, 400, 200):
        if t % period == 0 and bs % t == 0:
            return t
    return bs


def _body(x_ref, w_ref, b_ref, g_ref, bt_ref, p_ref, o_ref,
          h_s, sum_s, sq_s, ss_s, *, tile_m, cnt):
    ph = pl.program_id(0)
    i = pl.program_id(1)
    off = pl.multiple_of(i * tile_m, tile_m)

    @pl.when(ph == 0)
    def _compute():
        @pl.when(i == 0)
        def _init():
            sum_s[...] = jnp.zeros_like(sum_s)
            sq_s[...] = jnp.zeros_like(sq_s)

        h = jnp.dot(x_ref[...].astype(jnp.bfloat16), w_ref[...],
                    preferred_element_type=jnp.float32)
        h = h + b_ref[...]
        h_s[pl.ds(off, tile_m), :] = h
        sum_s[...] += jnp.sum(h, axis=1, keepdims=True)
        sq_s[...] += jnp.sum(h * h, axis=1, keepdims=True)

    @pl.when(ph == 1)
    def _apply():
        @pl.when(i == 0)
        def _finalize():
            proj = p_ref[...]
            dn = (((0,), (0,)), ((), ()))
            s_ch = jax.lax.dot_general(proj, sum_s[...], dn,
                                       preferred_element_type=jnp.float32)
            q_ch = jax.lax.dot_general(proj, sq_s[...], dn,
                                       preferred_element_type=jnp.float32)
            mean = s_ch * (1.0 / cnt)
            var = jnp.maximum(q_ch * (1.0 / cnt) - mean * mean, 0.0)
            scale = g_ref[...] * jax.lax.rsqrt(var + _EPS)
            shift = bt_ref[...] - mean * scale
            ss_s[:, 0:1] = jnp.dot(proj, scale, preferred_element_type=jnp.float32)
            ss_s[:, 1:2] = jnp.dot(proj, shift, preferred_element_type=jnp.float32)

        h = h_s[pl.ds(off, tile_m), :]
        y = h * ss_s[:, 0:1] + ss_s[:, 1:2]
        o_ref[...] = jnp.maximum(y, 0.0).astype(o_ref.dtype)


def kernel(x, w, b, gamma, beta):
    bs, c, d = x.shape
    d1 = c * d
    d2 = w.shape[1]
    nch = gamma.shape[0]
    n_grp = bs // nch
    cnt = float(n_grp * d2)

    x2 = x.reshape(bs, d1)
    wb = w.astype(jnp.bfloat16)
    b2 = b.astype(jnp.float32).reshape(1, d2)
    g2 = gamma.astype(jnp.float32).reshape(nch, 1)
    bt2 = beta.astype(jnp.float32).reshape(nch, 1)

    tile_m = _pick_tile(bs, nch)
    n_tiles = bs // tile_m
    proj = jax.nn.one_hot(jnp.arange(tile_m) % nch, nch, dtype=jnp.float32)

    body = functools.partial(_body, tile_m=tile_m, cnt=cnt)

    y2 = pl.pallas_call(
        body,
        out_shape=jax.ShapeDtypeStruct((bs, d2), x.dtype),
        grid=(2, n_tiles),
        in_specs=[
            pl.BlockSpec((tile_m, d1),
                         lambda p, i: (i * (1 - p) + (n_tiles - 1) * p, 0)),
            pl.BlockSpec((d1, d2), lambda p, i: (0, 0)),
            pl.BlockSpec((1, d2), lambda p, i: (0, 0)),
            pl.BlockSpec((nch, 1), lambda p, i: (0, 0)),
            pl.BlockSpec((nch, 1), lambda p, i: (0, 0)),
            pl.BlockSpec((tile_m, nch), lambda p, i: (0, 0)),
        ],
        out_specs=pl.BlockSpec((tile_m, d2), lambda p, i: (i * p, 0)),
        scratch_shapes=[
            pltpu.VMEM((bs, d2), jnp.float32),
            pltpu.VMEM((tile_m, 1), jnp.float32),
            pltpu.VMEM((tile_m, 1), jnp.float32),
            pltpu.VMEM((tile_m, 2), jnp.float32),
        ],
        compiler_params=pltpu.CompilerParams(
            dimension_semantics=("arbitrary", "arbitrary"),
            vmem_limit_bytes=int(60 * 1024 * 1024)),
        cost_estimate=pl.CostEstimate(
            flops=2 * bs * d1 * d2 + 6 * bs * d2,
            transcendentals=0,
            bytes_accessed=(bs * d1 + bs * d2 + d1 * d2) * 4),
    )(x2, wb, b2, g2, bt2, proj)

    return y2.reshape(n_grp, nch, d2)
